# baseline (speedup 1.0000x reference)
import jax
import jax.numpy as jnp
from jax.experimental import pallas as pl
from jax.experimental.pallas import tpu as pltpu

V = 10000
B = 64
KCHEB = 5
F = 16
POOL = 8
VP = V // POOL
VPP = 1280
RA = 400
RB = 1024
NB4 = 10
CH = 64

_MM = (((1,), (0,)), ((), ()))
_MMT = (((1,), (1,)), ((), ()))
RT = 1024


def _xpose_body(x_ref, xc_ref, xb_ref):
    xt = x_ref[...].T
    xc_ref[...] = xt
    xb_ref[...] = xt.astype(jnp.bfloat16)


def _xpose(x_in):
    return pl.pallas_call(
        _xpose_body,
        grid=(pl.cdiv(V, RT),),
        in_specs=[pl.BlockSpec((B, RT), lambda i: (0, i))],
        out_specs=[
            pl.BlockSpec((RT, B), lambda i: (i, 0)),
            pl.BlockSpec((RT, B), lambda i: (i, 0)),
        ],
        out_shape=[
            jax.ShapeDtypeStruct((V, B), jnp.float32),
            jax.ShapeDtypeStruct((V, B), jnp.bfloat16),
        ],
    )(x_in)


def _pass1_body(L_ref, xb_ref, x_blk_ref, Lb_ref, y1_ref, y1b_ref):
    Lb = L_ref[...].astype(jnp.bfloat16)
    Lb_ref[...] = Lb
    acc = jax.lax.dot_general(Lb, xb_ref[...], _MM,
                              preferred_element_type=jnp.float32)
    y1 = acc - x_blk_ref[...]
    y1_ref[...] = y1
    y1b_ref[...] = y1.astype(jnp.bfloat16)


def _pass1(L0, xbc, x0c):
    return pl.pallas_call(
        _pass1_body,
        grid=(V // RA,),
        in_specs=[
            pl.BlockSpec((RA, V), lambda i: (i, 0)),
            pl.BlockSpec(memory_space=pltpu.VMEM),
            pl.BlockSpec((RA, B), lambda i: (i, 0)),
        ],
        out_specs=[
            pl.BlockSpec((RA, V), lambda i: (i, 0)),
            pl.BlockSpec((RA, B), lambda i: (i, 0)),
            pl.BlockSpec((RA, B), lambda i: (i, 0)),
        ],
        out_shape=[
            jax.ShapeDtypeStruct((V, V), jnp.bfloat16),
            jax.ShapeDtypeStruct((V, B), jnp.float32),
            jax.ShapeDtypeStruct((V, B), jnp.bfloat16),
        ],
    )(L0, xbc, x0c)


def _cheby_body(Lb_ref, curb_ref, cur_blk_ref, prev_blk_ref,
                out_ref, outb_ref):
    acc = jax.lax.dot_general(Lb_ref[...], curb_ref[...], _MM,
                              preferred_element_type=jnp.float32)
    y = 2.0 * (acc - cur_blk_ref[...]) - prev_blk_ref[...]
    out_ref[...] = y
    outb_ref[...] = y.astype(jnp.bfloat16)


def _cheby(Lb, curb, cur, prev):
    return pl.pallas_call(
        _cheby_body,
        grid=(pl.cdiv(V, RB),),
        in_specs=[
            pl.BlockSpec((RB, V), lambda i: (i, 0)),
            pl.BlockSpec(memory_space=pltpu.VMEM),
            pl.BlockSpec((RB, B), lambda i: (i, 0)),
            pl.BlockSpec((RB, B), lambda i: (i, 0)),
        ],
        out_specs=[
            pl.BlockSpec((RB, B), lambda i: (i, 0)),
            pl.BlockSpec((RB, B), lambda i: (i, 0)),
        ],
        out_shape=[
            jax.ShapeDtypeStruct((V, B), jnp.float32),
            jax.ShapeDtypeStruct((V, B), jnp.bfloat16),
        ],
    )(Lb, curb, cur, prev)


def _pass4_body(Lb_ref, y3b_ref, y3_blk_ref, y2_blk_ref,
                y1_blk_ref, y0_blk_ref, wcl_ref, bcl_ref, wf1_ref,
                hout_ref, y4s_ref, pools_ref, htacc_ref):
    i = pl.program_id(0)
    acc = jax.lax.dot_general(Lb_ref[...], y3b_ref[...], _MM,
                              preferred_element_type=jnp.float32)
    y4s_ref[...] = 2.0 * (acc - y3_blk_ref[...]) - y2_blk_ref[...]

    wcl = wcl_ref[...]
    bcl = bcl_ref[...]
    refs = (y0_blk_ref, y1_blk_ref, y2_blk_ref, y3_blk_ref, y4s_ref)
    for c in range(RB // CH):
        base = c * CH
        ycs = [r[pl.ds(base, CH), :] for r in refs]
        for f in range(F):
            a = ycs[0] * wcl[f, 0]
            for k in range(1, KCHEB):
                a = a + ycs[k] * wcl[f, k]
            a = a + bcl[0, f]
            pooled = jnp.max(a.reshape(CH // POOL, POOL, B), axis=1)
            pools_ref[f, pl.ds(c * (CH // POOL), CH // POOL), :] = pooled

    @pl.when(i == 0)
    def _():
        htacc_ref[...] = jnp.zeros((128, B), jnp.float32)

    rows = jax.lax.broadcasted_iota(jnp.int32, (RB // POOL, B), 0)
    valid = rows < (VP - i * (RB // POOL))
    hdelta = jnp.zeros((128, B), jnp.float32)
    for f in range(F):
        pm = jnp.where(valid, jnp.maximum(pools_ref[f], 0.0), 0.0)
        hdelta = hdelta + jax.lax.dot_general(
            wf1_ref[f], pm.astype(jnp.bfloat16), _MM,
            preferred_element_type=jnp.float32)
    htacc_ref[...] += hdelta

    @pl.when(i == NB4 - 1)
    def _():
        hout_ref[...] = htacc_ref[...]


def _pass4(Lb, y3b, y3, y2, y1, y0, wcl, bcl, Wf1r):
    return pl.pallas_call(
        _pass4_body,
        grid=(NB4,),
        in_specs=[
            pl.BlockSpec((RB, V), lambda i: (i, 0)),
            pl.BlockSpec(memory_space=pltpu.VMEM),
            pl.BlockSpec((RB, B), lambda i: (i, 0)),
            pl.BlockSpec((RB, B), lambda i: (i, 0)),
            pl.BlockSpec((RB, B), lambda i: (i, 0)),
            pl.BlockSpec((RB, B), lambda i: (i, 0)),
            pl.BlockSpec(memory_space=pltpu.VMEM),
            pl.BlockSpec(memory_space=pltpu.VMEM),
            pl.BlockSpec((F, 128, RB // POOL), lambda i: (0, 0, i)),
        ],
        out_specs=pl.BlockSpec((128, B), lambda i: (0, 0)),
        out_shape=jax.ShapeDtypeStruct((128, B), jnp.float32),
        scratch_shapes=[
            pltpu.VMEM((RB, B), jnp.float32),
            pltpu.VMEM((F, RB // POOL, B), jnp.float32),
            pltpu.VMEM((128, B), jnp.float32),
        ],
    )(Lb, y3b, y3, y2, y1, y0, wcl, bcl, Wf1r)


def _fc_body(hacc_ref, bf1_ref, wf2_ref, bf2_ref, wf3_ref, bf3_ref,
             xb_ref, wn1_ref, bn1_ref, wn2_ref, bn2_ref, ws_ref, bs_ref,
             dec_ref, hid_ref, out_ref):
    ht = jnp.maximum(hacc_ref[...] + bf1_ref[...], 0.0)
    h = ht.T
    hid_ref[...] = h
    xdt = jnp.maximum(
        jax.lax.dot_general(wf2_ref[...], ht, _MM,
                            preferred_element_type=jnp.float32)
        + bf2_ref[...], 0.0)
    dec_ref[...] = jax.lax.dot_general(
        xdt.T, wf3_ref[...], _MMT,
        preferred_element_type=jnp.float32) + bf3_ref[...]
    xn = jnp.maximum(
        jax.lax.dot_general(xb_ref[...].astype(jnp.bfloat16),
                            wn1_ref[...], _MMT,
                            preferred_element_type=jnp.float32)
        + bn1_ref[...], 0.0)
    xn = jnp.maximum(
        jax.lax.dot_general(xn.astype(jnp.bfloat16), wn2_ref[...], _MMT,
                            preferred_element_type=jnp.float32)
        + bn2_ref[...], 0.0)
    z = jnp.concatenate([h, xn], axis=1)
    logits = jax.lax.dot_general(
        z, ws_ref[...], _MMT,
        preferred_element_type=jnp.float32) + bs_ref[...]
    m = jnp.max(logits, axis=1, keepdims=True)
    e = logits - m
    out_ref[...] = e - jnp.log(jnp.sum(jnp.exp(e), axis=1, keepdims=True))


def kernel(x_in, d, L, W_cl1, b_cl1, W_fc1, b_fc1, W_fc2, b_fc2,
           W_fc3, b_fc3, W_nn1, b_nn1, W_nn2, b_nn2, W_sum2, b_sum2):
    L0 = L[0]
    x0c, xbc = _xpose(x_in)
    Lb, y1, y1b = _pass1(L0, xbc, x0c)
    y2, y2b = _cheby(Lb, y1b, y1, x0c)
    y3, y3b = _cheby(Lb, y2b, y2, y1)
    Wf1r = W_fc1.reshape(128, VP, F).transpose(2, 0, 1).astype(jnp.bfloat16)
    Wf1r = jnp.concatenate(
        [Wf1r, jnp.zeros((F, 128, VPP - VP), jnp.bfloat16)], axis=2)
    hacc = _pass4(Lb, y3b, y3, y2, y1, x0c, W_cl1,
                  b_cl1.reshape(1, F), Wf1r)
    dec, hid, out = pl.pallas_call(
        _fc_body,
        out_shape=[
            jax.ShapeDtypeStruct((B, V), jnp.float32),
            jax.ShapeDtypeStruct((B, 128), jnp.float32),
            jax.ShapeDtypeStruct((B, 10), jnp.float32),
        ],
    )(hacc, b_fc1.reshape(128, 1), W_fc2, b_fc2.reshape(64, 1),
      W_fc3, b_fc3.reshape(1, V), x_in, W_nn1.astype(jnp.bfloat16),
      b_nn1.reshape(1, 256), W_nn2, b_nn2.reshape(1, 128),
      W_sum2, b_sum2.reshape(1, 10))
    return dec, hid, out
